# Initial kernel scaffold; baseline (speedup 1.0000x reference)
import jax
import jax.numpy as jnp
from jax.experimental import pallas as pl


def kernel(atom_ftr, bond_ftr, massive, edge_index, mol_ids, W_init_v, b_init_v, W_init_e, b_init_e, W_p, W_q, W_msg, W_upd_v, W_upd_e, W_ham, W_att, W_fp, W_c1, b_c1, W_c2, b_c2):
    raise NotImplementedError("write your pallas kernel here")


# speedup vs baseline: 1.0611x; 1.0611x over previous
import functools
import jax
import jax.numpy as jnp
from jax import lax
from jax.experimental import pallas as pl
from jax.experimental.pallas import tpu as pltpu

N = 10000
E = 160000
HV = 128
HE = 64
PQ = 3
ME = 64
MM = 128
NM = 64
TAU = 0.25

BN = 1000
BE = 2000


def _row_spec(block_rows, cols):
    return pl.BlockSpec((block_rows, cols), lambda i: (i, 0))


def _full_spec(shape):
    return pl.BlockSpec(shape, lambda *a: tuple(0 for _ in shape))


def _init_nodes_body(atom_ref, wiv_ref, biv_ref, wp_ref, wq_ref,
                     wmu_ref, wmv_ref, wd_ref, whu_ref, whv_ref,
                     hv_ref, p_ref, q_ref, tu_ref, tv_ref):
    hv = jnp.tanh(jnp.dot(atom_ref[...], wiv_ref[...],
                          preferred_element_type=jnp.float32) + biv_ref[...])
    hv_ref[...] = hv
    p = jnp.dot(hv, wp_ref[...], preferred_element_type=jnp.float32)
    q = jnp.dot(hv, wq_ref[...], preferred_element_type=jnp.float32)
    p_ref[...] = p
    q_ref[...] = q
    qd = jnp.dot(q, wd_ref[...], preferred_element_type=jnp.float32)
    a = jnp.dot(hv, wmu_ref[...], preferred_element_type=jnp.float32) + qd
    b = jnp.dot(hv, wmv_ref[...], preferred_element_type=jnp.float32) - qd
    cu = jnp.dot(hv, whu_ref[...], preferred_element_type=jnp.float32)
    cv = jnp.dot(hv, whv_ref[...], preferred_element_type=jnp.float32)
    z = jnp.zeros((a.shape[0], 15), jnp.float32)
    tu_ref[...] = jnp.concatenate([a, cu, z], axis=1)
    tv_ref[...] = jnp.concatenate([b, cv, z], axis=1)


def _init_nodes(atom_ftr, wiv, biv, wp8, wq8, wmu, wmv, wd8, whu, whv):
    return pl.pallas_call(
        _init_nodes_body,
        grid=(N // BN,),
        in_specs=[
            _row_spec(BN, HV), _full_spec((HV, HV)), _full_spec((1, HV)),
            _full_spec((HV, 8)), _full_spec((HV, 8)),
            _full_spec((HV, ME)), _full_spec((HV, ME)), _full_spec((8, ME)),
            _full_spec((HV, 1)), _full_spec((HV, 1)),
        ],
        out_specs=[
            _row_spec(BN, HV), _row_spec(BN, 8), _row_spec(BN, 8),
            _row_spec(BN, 80), _row_spec(BN, 80),
        ],
        out_shape=[
            jax.ShapeDtypeStruct((N, HV), jnp.float32),
            jax.ShapeDtypeStruct((N, 8), jnp.float32),
            jax.ShapeDtypeStruct((N, 8), jnp.float32),
            jax.ShapeDtypeStruct((N, 80), jnp.float32),
            jax.ShapeDtypeStruct((N, 80), jnp.float32),
        ],
    )(atom_ftr, wiv, biv, wp8, wq8, wmu, wmv, wd8, whu, whv)


def _node_tables_body(hv_ref, q_ref, wmu_ref, wmv_ref, wd_ref, whu_ref,
                      whv_ref, tu_ref, tv_ref):
    hv = hv_ref[...]
    qd = jnp.dot(q_ref[...], wd_ref[...], preferred_element_type=jnp.float32)
    a = jnp.dot(hv, wmu_ref[...], preferred_element_type=jnp.float32) + qd
    b = jnp.dot(hv, wmv_ref[...], preferred_element_type=jnp.float32) - qd
    cu = jnp.dot(hv, whu_ref[...], preferred_element_type=jnp.float32)
    cv = jnp.dot(hv, whv_ref[...], preferred_element_type=jnp.float32)
    z = jnp.zeros((a.shape[0], 15), jnp.float32)
    tu_ref[...] = jnp.concatenate([a, cu, z], axis=1)
    tv_ref[...] = jnp.concatenate([b, cv, z], axis=1)


def _node_tables(hv, q8, wmu, wmv, wd8, whu, whv):
    return pl.pallas_call(
        _node_tables_body,
        grid=(N // BN,),
        in_specs=[
            _row_spec(BN, HV), _row_spec(BN, 8),
            _full_spec((HV, ME)), _full_spec((HV, ME)), _full_spec((8, ME)),
            _full_spec((HV, 1)), _full_spec((HV, 1)),
        ],
        out_specs=[_row_spec(BN, 80), _row_spec(BN, 80)],
        out_shape=[
            jax.ShapeDtypeStruct((N, 80), jnp.float32),
            jax.ShapeDtypeStruct((N, 80), jnp.float32),
        ],
    )(hv, q8, wmu, wmv, wd8, whu, whv)


def _init_edges_body(bond_ref, wie_ref, bie_ref, wme_ref, whe_ref,
                     he_ref, mhe_ref, ce_ref):
    he = jnp.tanh(jnp.dot(bond_ref[...], wie_ref[...],
                          preferred_element_type=jnp.float32) + bie_ref[...])
    he_ref[...] = he
    mhe_ref[...] = jnp.dot(he, wme_ref[...], preferred_element_type=jnp.float32)
    ce_ref[...] = jnp.dot(he, whe_ref[...], preferred_element_type=jnp.float32)


def _init_edges(bond_ftr, wie, bie, wme, whe):
    return pl.pallas_call(
        _init_edges_body,
        grid=(E // BE,),
        in_specs=[
            _row_spec(BE, 16), _full_spec((16, HE)), _full_spec((1, HE)),
            _full_spec((HE, ME)), _full_spec((HE, 1)),
        ],
        out_specs=[_row_spec(BE, HE), _row_spec(BE, ME), _row_spec(BE, 1)],
        out_shape=[
            jax.ShapeDtypeStruct((E, HE), jnp.float32),
            jax.ShapeDtypeStruct((E, ME), jnp.float32),
            jax.ShapeDtypeStruct((E, 1), jnp.float32),
        ],
    )(bond_ftr, wie, bie, wme, whe)


def _upd_v_body(hv_ref, agg_ref, wv1_ref, wv2_ref, out_ref):
    agg = agg_ref[0] + agg_ref[1]
    out_ref[...] = jax.nn.relu(
        jnp.dot(hv_ref[...], wv1_ref[...], preferred_element_type=jnp.float32)
        + jnp.dot(agg, wv2_ref[...], preferred_element_type=jnp.float32))


def _upd_v(hv, agg2, wv1, wv2):
    return pl.pallas_call(
        _upd_v_body,
        grid=(N // BN,),
        in_specs=[
            _row_spec(BN, HV),
            pl.BlockSpec((2, BN, ME), lambda i: (0, i, 0)),
            _full_spec((HV, HV)), _full_spec((ME, HV)),
        ],
        out_specs=_row_spec(BN, HV),
        out_shape=jax.ShapeDtypeStruct((N, HV), jnp.float32),
    )(hv, agg2, wv1, wv2)


def _upd_e_proj_body(he_ref, m_ref, wa_ref, wb_ref, wc_ref, wd_ref,
                     mhe_ref, ce_ref):
    t_he = jax.nn.relu(
        jnp.dot(he_ref[...], wa_ref[...], preferred_element_type=jnp.float32)
        + jnp.dot(m_ref[...], wb_ref[...], preferred_element_type=jnp.float32))
    mhe_ref[...] = jnp.dot(t_he, wc_ref[...], preferred_element_type=jnp.float32)
    ce_ref[...] = jnp.dot(t_he, wd_ref[...], preferred_element_type=jnp.float32)


def _upd_e_proj(he, m, wa, wb, wc, wd):
    return pl.pallas_call(
        _upd_e_proj_body,
        grid=(E // BE,),
        in_specs=[
            _row_spec(BE, HE), _row_spec(BE, ME),
            _full_spec((HE, HE)), _full_spec((ME, HE)),
            _full_spec((HE, ME)), _full_spec((HE, 1)),
        ],
        out_specs=[_row_spec(BE, ME), _row_spec(BE, 1)],
        out_shape=[
            jax.ShapeDtypeStruct((E, ME), jnp.float32),
            jax.ShapeDtypeStruct((E, 1), jnp.float32),
        ],
    )(he, m, wa, wb, wc, wd)


def _ham_update_body(q_ref, p_ref, f_ref, mass_ref, qo_ref, po_ref):
    f = f_ref[0] + f_ref[1]
    q = q_ref[...] + TAU * p_ref[...] / mass_ref[...]
    qo_ref[...] = q
    po_ref[...] = (1.0 - 0.1 * TAU) * p_ref[...] + TAU * f


def _ham_update(q8, p8, f2, mass):
    return pl.pallas_call(
        _ham_update_body,
        grid=(N // BN,),
        in_specs=[
            _row_spec(BN, 8), _row_spec(BN, 8),
            pl.BlockSpec((2, BN, 8), lambda i: (0, i, 0)),
            _row_spec(BN, 1),
        ],
        out_specs=[_row_spec(BN, 8), _row_spec(BN, 8)],
        out_shape=[
            jax.ShapeDtypeStruct((N, 8), jnp.float32),
            jax.ShapeDtypeStruct((N, 8), jnp.float32),
        ],
    )(q8, p8, f2, mass)


def _readout_body(hv_ref, p_ref, q_ref, mid_ref, wfp_ref, wahv_ref, wap_ref,
                  waq_ref, wc1_ref, bc1_ref, wc2_ref, bc2_ref,
                  fp_ref, conf_ref):
    hv = hv_ref[...]
    hvp = jnp.dot(hv, wfp_ref[...], preferred_element_type=jnp.float32)
    a = jax.nn.sigmoid(
        jnp.dot(hv, wahv_ref[...], preferred_element_type=jnp.float32)
        + jnp.dot(p_ref[...], wap_ref[...], preferred_element_type=jnp.float32)
        + jnp.dot(q_ref[...], waq_ref[...], preferred_element_type=jnp.float32))
    mids = mid_ref[...]
    oh = (mids == lax.broadcasted_iota(jnp.int32, (N, NM), 1)).astype(jnp.float32)
    hm = lax.dot_general(oh, a * hvp, (((0,), (0,)), ((), ())),
                         preferred_element_type=jnp.float32)
    for _ in range(2):
        g = jnp.dot(oh, hm, preferred_element_type=jnp.float32)
        a2 = jax.nn.sigmoid(jnp.sum(hvp * g, axis=1, keepdims=True))
        hm = lax.dot_general(oh, a2 * hvp, (((0,), (0,)), ((), ())),
                             preferred_element_type=jnp.float32)
    fp_ref[...] = hm
    conf_ref[...] = (
        jnp.dot(jax.nn.relu(
            jnp.dot(q_ref[...], wc1_ref[...], preferred_element_type=jnp.float32)
            + bc1_ref[...]), wc2_ref[...], preferred_element_type=jnp.float32)
        + bc2_ref[...])


def _readout(hv, p8, q8, mid, wfp, wahv, wap8, waq8, wc18, bc1, wc2, bc38):
    return pl.pallas_call(
        _readout_body,
        in_specs=[
            _full_spec((N, HV)), _full_spec((N, 8)), _full_spec((N, 8)),
            _full_spec((N, 1)), _full_spec((HV, MM)), _full_spec((HV, 1)),
            _full_spec((8, 1)), _full_spec((8, 1)),
            _full_spec((8, MM)), _full_spec((1, MM)),
            _full_spec((MM, 8)), _full_spec((1, 8)),
        ],
        out_specs=[_full_spec((NM, MM)), _full_spec((N, 8))],
        out_shape=[
            jax.ShapeDtypeStruct((NM, MM), jnp.float32),
            jax.ShapeDtypeStruct((N, 8), jnp.float32),
        ],
    )(hv, p8, q8, mid, wfp, wahv, wap8, waq8, wc18, bc1, wc2, bc38)


def _edge_msg(tu, tv, mhe, ce, u, v, need_m):
    au = tu[u]
    bv = tv[v]
    m = jax.nn.relu(au[:, :ME] + bv[:, :ME] + mhe)
    coef = jnp.tanh(au[:, ME] + bv[:, ME] + ce[:, 0])
    agg = jnp.zeros((N, ME), jnp.float32).at[v].add(m)
    return m, coef, jnp.stack([agg, jnp.zeros_like(agg)])


def _ham_f(q8, coef, u, v):
    dq = q8[v] - q8[u]
    f = jnp.zeros((N, 8), jnp.float32).at[u].add(coef[:, None] * dq)
    return jnp.stack([f, jnp.zeros_like(f)])


def kernel(atom_ftr, bond_ftr, massive, edge_index, mol_ids,
           W_init_v, b_init_v, W_init_e, b_init_e, W_p, W_q, W_msg,
           W_upd_v, W_upd_e, W_ham, W_att, W_fp, W_c1, b_c1, W_c2, b_c2):
    u = edge_index[0]
    v = edge_index[1]

    def pad_cols(w, cols):
        return jnp.pad(w, ((0, 0), (0, cols - w.shape[1])))

    def pad_rows(w, rows):
        return jnp.pad(w, ((0, rows - w.shape[0]), (0, 0)))

    wp8 = pad_cols(W_p, 8)
    wq8 = pad_cols(W_q, 8)
    wd8 = [pad_rows(W_msg[i][2 * HV + HE:], 8) for i in range(2)]
    wmu = [W_msg[i][:HV] for i in range(2)]
    wmv = [W_msg[i][HV:2 * HV] for i in range(2)]
    whu = [W_ham[i][:HV] for i in range(2)]
    whv = [W_ham[i][HV:2 * HV] for i in range(2)]

    hv, p8, q8, tu, tv = _init_nodes(
        atom_ftr, W_init_v, b_init_v[None], wp8, wq8,
        wmu[0], wmv[0], wd8[0], whu[0], whv[0])
    he, mhe, ce = _init_edges(
        bond_ftr, W_init_e, b_init_e[None],
        W_msg[0][2 * HV:2 * HV + HE], W_ham[0][2 * HV:])

    for i in range(2):
        m, coef, agg2 = _edge_msg(tu, tv, mhe, ce, u, v, need_m=(i == 0))
        t_hv = _upd_v(hv, agg2, W_upd_v[i][:HV], W_upd_v[i][HV:])
        if i == 0:
            mhe, ce = _upd_e_proj(he, m, W_upd_e[0][:HE], W_upd_e[0][HE:],
                                  W_msg[1][2 * HV:2 * HV + HE],
                                  W_ham[1][2 * HV:])
        for _ in range(4):
            f2 = _ham_f(q8, coef, u, v)
            q8, p8 = _ham_update(q8, p8, f2, massive)
        hv = t_hv
        if i == 0:
            tu, tv = _node_tables(hv, q8, wmu[1], wmv[1], wd8[1],
                                  whu[1], whv[1])

    fp, conf8 = _readout(
        hv, p8, q8, mol_ids[:, None].astype(jnp.int32),
        W_fp, W_att[:HV], pad_rows(W_att[HV:HV + PQ], 8),
        pad_rows(W_att[HV + PQ:], 8), pad_rows(W_c1, 8), b_c1[None],
        pad_cols(W_c2, 8), pad_cols(b_c2[None], 8))
    return (fp, conf8[:, :PQ])


# speedup vs baseline: 3.5505x; 3.3460x over previous
import functools
import jax
import jax.numpy as jnp
from jax import lax
from jax.experimental import pallas as pl
from jax.experimental.pallas import tpu as pltpu
from jax.experimental.pallas import tpu_sc as plsc

N = 10000
E = 160000
HV = 128
HE = 64
PQ = 3
ME = 64
MM = 128
NM = 64
TAU = 0.25

BN = 1000
BE = 2000

NP = 10112
E2 = 163840
NW = 32
CHUNK = 128
NCH = E2 // (NW * CHUNK)
STRIPE = NP // 16


def _row_spec(block_rows, cols):
    return pl.BlockSpec((block_rows, cols), lambda i: (i, 0))


def _full_spec(shape):
    return pl.BlockSpec(shape, lambda *a: tuple(0 for _ in shape))


def _init_nodes_body(atom_ref, wiv_ref, biv_ref, wp_ref, wq_ref,
                     wmu_ref, wmv_ref, wd_ref, whu_ref, whv_ref,
                     hv_ref, p_ref, q_ref, tu_ref, tv_ref):
    hv = jnp.tanh(jnp.dot(atom_ref[...], wiv_ref[...],
                          preferred_element_type=jnp.float32) + biv_ref[...])
    hv_ref[...] = hv
    p = jnp.dot(hv, wp_ref[...], preferred_element_type=jnp.float32)
    q = jnp.dot(hv, wq_ref[...], preferred_element_type=jnp.float32)
    p_ref[...] = p
    q_ref[...] = q
    qd = jnp.dot(q, wd_ref[...], preferred_element_type=jnp.float32)
    a = jnp.dot(hv, wmu_ref[...], preferred_element_type=jnp.float32) + qd
    b = jnp.dot(hv, wmv_ref[...], preferred_element_type=jnp.float32) - qd
    cu = jnp.dot(hv, whu_ref[...], preferred_element_type=jnp.float32)
    cv = jnp.dot(hv, whv_ref[...], preferred_element_type=jnp.float32)
    tu_ref[...] = jnp.concatenate(
        [a, jnp.broadcast_to(cu, (a.shape[0], 16))], axis=1)
    tv_ref[...] = jnp.concatenate(
        [b, jnp.broadcast_to(cv, (b.shape[0], 16))], axis=1)


def _init_nodes(atom_ftr, wiv, biv, wp8, wq8, wmu, wmv, wd8, whu, whv):
    return pl.pallas_call(
        _init_nodes_body,
        grid=(N // BN,),
        in_specs=[
            _row_spec(BN, HV), _full_spec((HV, HV)), _full_spec((1, HV)),
            _full_spec((HV, 16)), _full_spec((HV, 16)),
            _full_spec((HV, ME)), _full_spec((HV, ME)), _full_spec((16, ME)),
            _full_spec((HV, 1)), _full_spec((HV, 1)),
        ],
        out_specs=[
            _row_spec(BN, HV), _row_spec(BN, 16), _row_spec(BN, 16),
            _row_spec(BN, 80), _row_spec(BN, 80),
        ],
        out_shape=[
            jax.ShapeDtypeStruct((N, HV), jnp.float32),
            jax.ShapeDtypeStruct((N, 16), jnp.float32),
            jax.ShapeDtypeStruct((N, 16), jnp.float32),
            jax.ShapeDtypeStruct((N, 80), jnp.float32),
            jax.ShapeDtypeStruct((N, 80), jnp.float32),
        ],
    )(atom_ftr, wiv, biv, wp8, wq8, wmu, wmv, wd8, whu, whv)


def _node_tables_body(hv_ref, q_ref, wmu_ref, wmv_ref, wd_ref, whu_ref,
                      whv_ref, tu_ref, tv_ref):
    hv = hv_ref[...]
    qd = jnp.dot(q_ref[...], wd_ref[...], preferred_element_type=jnp.float32)
    a = jnp.dot(hv, wmu_ref[...], preferred_element_type=jnp.float32) + qd
    b = jnp.dot(hv, wmv_ref[...], preferred_element_type=jnp.float32) - qd
    cu = jnp.dot(hv, whu_ref[...], preferred_element_type=jnp.float32)
    cv = jnp.dot(hv, whv_ref[...], preferred_element_type=jnp.float32)
    tu_ref[...] = jnp.concatenate(
        [a, jnp.broadcast_to(cu, (a.shape[0], 16))], axis=1)
    tv_ref[...] = jnp.concatenate(
        [b, jnp.broadcast_to(cv, (b.shape[0], 16))], axis=1)


def _node_tables(hv, q8, wmu, wmv, wd8, whu, whv):
    return pl.pallas_call(
        _node_tables_body,
        grid=(N // BN,),
        in_specs=[
            _row_spec(BN, HV), _row_spec(BN, 16),
            _full_spec((HV, ME)), _full_spec((HV, ME)), _full_spec((16, ME)),
            _full_spec((HV, 1)), _full_spec((HV, 1)),
        ],
        out_specs=[_row_spec(BN, 80), _row_spec(BN, 80)],
        out_shape=[
            jax.ShapeDtypeStruct((N, 80), jnp.float32),
            jax.ShapeDtypeStruct((N, 80), jnp.float32),
        ],
    )(hv, q8, wmu, wmv, wd8, whu, whv)


def _init_edges_body(bond_ref, wie_ref, bie_ref, wme_ref, whe_ref,
                     he_ref, mhe_ref, ce_ref):
    he = jnp.tanh(jnp.dot(bond_ref[...], wie_ref[...],
                          preferred_element_type=jnp.float32) + bie_ref[...])
    he_ref[...] = he
    mhe_ref[...] = jnp.dot(he, wme_ref[...], preferred_element_type=jnp.float32)
    ce = jnp.dot(he, whe_ref[...], preferred_element_type=jnp.float32)
    ce_ref[...] = jnp.broadcast_to(ce, (ce.shape[0], 16))


def _init_edges(bond_ftr, wie, bie, wme, whe):
    return pl.pallas_call(
        _init_edges_body,
        grid=(E // BE,),
        in_specs=[
            _row_spec(BE, 16), _full_spec((16, HE)), _full_spec((1, HE)),
            _full_spec((HE, ME)), _full_spec((HE, 1)),
        ],
        out_specs=[_row_spec(BE, HE), _row_spec(BE, ME), _row_spec(BE, 16)],
        out_shape=[
            jax.ShapeDtypeStruct((E, HE), jnp.float32),
            jax.ShapeDtypeStruct((E, ME), jnp.float32),
            jax.ShapeDtypeStruct((E, 16), jnp.float32),
        ],
    )(bond_ftr, wie, bie, wme, whe)


def _upd_v_body(hv_ref, agg_ref, wv1_ref, wv2_ref, out_ref):
    agg = agg_ref[0] + agg_ref[1]
    out_ref[...] = jax.nn.relu(
        jnp.dot(hv_ref[...], wv1_ref[...], preferred_element_type=jnp.float32)
        + jnp.dot(agg, wv2_ref[...], preferred_element_type=jnp.float32))


def _upd_v(hv, agg2, wv1, wv2):
    return pl.pallas_call(
        _upd_v_body,
        grid=(N // BN,),
        in_specs=[
            _row_spec(BN, HV),
            pl.BlockSpec((2, BN, ME), lambda i: (0, i, 0)),
            _full_spec((HV, HV)), _full_spec((ME, HV)),
        ],
        out_specs=_row_spec(BN, HV),
        out_shape=jax.ShapeDtypeStruct((N, HV), jnp.float32),
    )(hv, agg2, wv1, wv2)


def _upd_e_proj_body(he_ref, m_ref, wa_ref, wb_ref, wc_ref, wd_ref,
                     mhe_ref, ce_ref):
    t_he = jax.nn.relu(
        jnp.dot(he_ref[...], wa_ref[...], preferred_element_type=jnp.float32)
        + jnp.dot(m_ref[...], wb_ref[...], preferred_element_type=jnp.float32))
    mhe_ref[...] = jnp.dot(t_he, wc_ref[...], preferred_element_type=jnp.float32)
    ce = jnp.dot(t_he, wd_ref[...], preferred_element_type=jnp.float32)
    ce_ref[...] = jnp.broadcast_to(ce, (ce.shape[0], 16))


def _upd_e_proj(he, m, wa, wb, wc, wd):
    return pl.pallas_call(
        _upd_e_proj_body,
        grid=(E // BE,),
        in_specs=[
            _row_spec(BE, HE), _row_spec(BE, ME),
            _full_spec((HE, HE)), _full_spec((ME, HE)),
            _full_spec((HE, ME)), _full_spec((HE, 1)),
        ],
        out_specs=[_row_spec(BE, ME), _row_spec(BE, 16)],
        out_shape=[
            jax.ShapeDtypeStruct((E, ME), jnp.float32),
            jax.ShapeDtypeStruct((E, 16), jnp.float32),
        ],
    )(he, m, wa, wb, wc, wd)


def _ham_update_body(q_ref, p_ref, f_ref, mass_ref, qo_ref, po_ref):
    f = f_ref[0] + f_ref[1]
    q = q_ref[...] + TAU * p_ref[...] / mass_ref[...]
    qo_ref[...] = q
    po_ref[...] = (1.0 - 0.1 * TAU) * p_ref[...] + TAU * f


def _ham_update(q16, p16, f2, mass):
    bn2 = NP // 4
    return pl.pallas_call(
        _ham_update_body,
        grid=(4,),
        in_specs=[
            _row_spec(bn2, 16), _row_spec(bn2, 16),
            pl.BlockSpec((2, bn2, 16), lambda i: (0, i, 0)),
            _row_spec(bn2, 1),
        ],
        out_specs=[_row_spec(bn2, 16), _row_spec(bn2, 16)],
        out_shape=[
            jax.ShapeDtypeStruct((NP, 16), jnp.float32),
            jax.ShapeDtypeStruct((NP, 16), jnp.float32),
        ],
    )(q16, p16, f2, mass)


def _readout_body(hv_ref, p_ref, q_ref, mid_ref, wfp_ref, wahv_ref, wap_ref,
                  waq_ref, wc1_ref, bc1_ref, wc2_ref, bc2_ref,
                  fp_ref, conf_ref):
    hv = hv_ref[...]
    hvp = jnp.dot(hv, wfp_ref[...], preferred_element_type=jnp.float32)
    a = jax.nn.sigmoid(
        jnp.dot(hv, wahv_ref[...], preferred_element_type=jnp.float32)
        + jnp.dot(p_ref[...], wap_ref[...], preferred_element_type=jnp.float32)
        + jnp.dot(q_ref[...], waq_ref[...], preferred_element_type=jnp.float32))
    mids = mid_ref[...]
    oh = (mids == lax.broadcasted_iota(jnp.int32, (N, NM), 1)).astype(jnp.float32)
    hm = lax.dot_general(oh, a * hvp, (((0,), (0,)), ((), ())),
                         preferred_element_type=jnp.float32)
    for _ in range(2):
        g = jnp.dot(oh, hm, preferred_element_type=jnp.float32)
        a2 = jax.nn.sigmoid(jnp.sum(hvp * g, axis=1, keepdims=True))
        hm = lax.dot_general(oh, a2 * hvp, (((0,), (0,)), ((), ())),
                             preferred_element_type=jnp.float32)
    fp_ref[...] = hm
    conf_ref[...] = (
        jnp.dot(jax.nn.relu(
            jnp.dot(q_ref[...], wc1_ref[...], preferred_element_type=jnp.float32)
            + bc1_ref[...]), wc2_ref[...], preferred_element_type=jnp.float32)
        + bc2_ref[...])


def _readout(hv, p8, q8, mid, wfp, wahv, wap8, waq8, wc18, bc1, wc2, bc38):
    return pl.pallas_call(
        _readout_body,
        in_specs=[
            _full_spec((N, HV)), _full_spec((N, 16)), _full_spec((N, 16)),
            _full_spec((N, 1)), _full_spec((HV, MM)), _full_spec((HV, 1)),
            _full_spec((16, 1)), _full_spec((16, 1)),
            _full_spec((16, MM)), _full_spec((1, MM)),
            _full_spec((MM, 8)), _full_spec((1, 8)),
        ],
        out_specs=[_full_spec((NM, MM)), _full_spec((N, 8))],
        out_shape=[
            jax.ShapeDtypeStruct((NM, MM), jnp.float32),
            jax.ShapeDtypeStruct((N, 8), jnp.float32),
        ],
    )(hv, p8, q8, mid, wfp, wahv, wap8, waq8, wc18, bc1, wc2, bc38)


_MESH = plsc.VectorSubcoreMesh(core_axis_name="c", subcore_axis_name="s")


def _copy_idx_row(src2d, j, dst1d):
    for k in range(CHUNK // 16):
        sl = pl.ds(k * 16, 16)
        dst1d[sl] = src2d[j, sl]


def _sc_edge_msg(tu, tv, mhe, ce, u2d, v2d, zeros64, write_m):
    out_type = [jax.ShapeDtypeStruct((E2, ME), jnp.float32)] if write_m else []
    out_type += [
        jax.ShapeDtypeStruct((E2, 16), jnp.float32),
        jax.ShapeDtypeStruct((2, NP, ME), jnp.float32),
    ]

    def body(tu_h, tv_h, mhe_h, ce_h, u2_h, v2_h, z_h, *refs):
        if write_m:
            m_out, coef_out, agg_out = refs[0], refs[1], refs[2]
            (uix, vix, u1, v1, au, bv, mhv, cev, mv, cfv, shared,
             s1, s2) = refs[3:]
        else:
            coef_out, agg_out = refs[0], refs[1]
            (uix, vix, u1, v1, au, bv, mhv, cev, mv, cfv, shared,
             s1, s2) = refs[2:]
        cid = lax.axis_index("c")
        sid = lax.axis_index("s")
        wid = cid * 16 + sid
        stripe = pl.ds(sid * STRIPE, STRIPE)
        pltpu.sync_copy(z_h.at[stripe], shared.at[stripe])
        pltpu.sync_copy(u2_h.at[pl.ds(wid * NCH, NCH)], uix)
        pltpu.sync_copy(v2_h.at[pl.ds(wid * NCH, NCH)], vix)
        plsc.subcore_barrier()

        def chunk(j, carry):
            base = wid * (NCH * CHUNK) + j * CHUNK
            _copy_idx_row(uix, j, u1)
            _copy_idx_row(vix, j, v1)
            cp1 = pltpu.async_copy(tu_h.at[u1], au, s1)
            cp2 = pltpu.async_copy(tv_h.at[v1], bv, s2)
            pltpu.sync_copy(mhe_h.at[pl.ds(base, CHUNK)], mhv)
            pltpu.sync_copy(ce_h.at[pl.ds(base, CHUNK)], cev)
            cp1.wait()
            cp2.wait()

            def row(r, c2):
                for cc in range(ME // 16):
                    sl = pl.ds(cc * 16, 16)
                    mv[r, sl] = jnp.maximum(au[r, sl] + bv[r, sl] + mhv[r, sl],
                                            0.0)
                tl = pl.ds(ME, 16)
                s = au[r, tl] + bv[r, tl] + cev[r, pl.ds(0, 16)]
                cfv[r, pl.ds(0, 16)] = 1.0 - 2.0 / (jnp.exp(2.0 * s) + 1.0)
                return c2

            lax.fori_loop(0, CHUNK, row, 0)
            if write_m:
                pltpu.sync_copy(mv, m_out.at[pl.ds(base, CHUNK)])
            pltpu.sync_copy(cfv, coef_out.at[pl.ds(base, CHUNK)])
            pltpu.sync_copy(mv, shared.at[v1], add=True)
            return carry

        lax.fori_loop(0, NCH, chunk, 0)
        plsc.subcore_barrier()
        pltpu.sync_copy(shared.at[stripe], agg_out.at[cid, stripe])

    scratch = [
        pltpu.VMEM((NCH, CHUNK), jnp.int32),
        pltpu.VMEM((NCH, CHUNK), jnp.int32),
        pltpu.VMEM((CHUNK,), jnp.int32),
        pltpu.VMEM((CHUNK,), jnp.int32),
        pltpu.VMEM((CHUNK, 80), jnp.float32),
        pltpu.VMEM((CHUNK, 80), jnp.float32),
        pltpu.VMEM((CHUNK, ME), jnp.float32),
        pltpu.VMEM((CHUNK, 16), jnp.float32),
        pltpu.VMEM((CHUNK, ME), jnp.float32),
        pltpu.VMEM((CHUNK, 16), jnp.float32),
        pltpu.VMEM_SHARED((NP, ME), jnp.float32),
        pltpu.SemaphoreType.DMA,
        pltpu.SemaphoreType.DMA,
    ]
    fn = pl.kernel(body, out_type=out_type, mesh=_MESH, scratch_types=scratch,
                   compiler_params=pltpu.CompilerParams(
                       use_tc_tiling_on_sc=False))
    outs = fn(tu, tv, mhe, ce, u2d, v2d, zeros64)
    if write_m:
        return outs[0], outs[1], outs[2]
    return None, outs[0], outs[1]


def _sc_ham_f(q16, coef, u2d, v2d, zeros16):
    out_type = [jax.ShapeDtypeStruct((2, NP, 16), jnp.float32)]

    def body(q_h, cf_h, u2_h, v2_h, z_h, f_out, uix, vix, u1, v1,
             qu, qv, wv, cfv, shared, s1, s2):
        cid = lax.axis_index("c")
        sid = lax.axis_index("s")
        wid = cid * 16 + sid
        stripe = pl.ds(sid * STRIPE, STRIPE)
        pltpu.sync_copy(z_h.at[stripe], shared.at[stripe])
        pltpu.sync_copy(u2_h.at[pl.ds(wid * NCH, NCH)], uix)
        pltpu.sync_copy(v2_h.at[pl.ds(wid * NCH, NCH)], vix)
        plsc.subcore_barrier()

        def chunk(j, carry):
            base = wid * (NCH * CHUNK) + j * CHUNK
            _copy_idx_row(uix, j, u1)
            _copy_idx_row(vix, j, v1)
            cp1 = pltpu.async_copy(q_h.at[u1], qu, s1)
            cp2 = pltpu.async_copy(q_h.at[v1], qv, s2)
            pltpu.sync_copy(cf_h.at[pl.ds(base, CHUNK)], cfv)
            cp1.wait()
            cp2.wait()

            def row(r, c2):
                sl = pl.ds(0, 16)
                wv[r, sl] = cfv[r, sl] * (qv[r, sl] - qu[r, sl])
                return c2

            lax.fori_loop(0, CHUNK, row, 0)
            pltpu.sync_copy(wv, shared.at[u1], add=True)
            return carry

        lax.fori_loop(0, NCH, chunk, 0)
        plsc.subcore_barrier()
        pltpu.sync_copy(shared.at[stripe], f_out.at[cid, stripe])

    scratch = [
        pltpu.VMEM((NCH, CHUNK), jnp.int32),
        pltpu.VMEM((NCH, CHUNK), jnp.int32),
        pltpu.VMEM((CHUNK,), jnp.int32),
        pltpu.VMEM((CHUNK,), jnp.int32),
        pltpu.VMEM((CHUNK, 16), jnp.float32),
        pltpu.VMEM((CHUNK, 16), jnp.float32),
        pltpu.VMEM((CHUNK, 16), jnp.float32),
        pltpu.VMEM((CHUNK, 16), jnp.float32),
        pltpu.VMEM_SHARED((NP, 16), jnp.float32),
        pltpu.SemaphoreType.DMA,
        pltpu.SemaphoreType.DMA,
    ]
    fn = pl.kernel(body, out_type=out_type, mesh=_MESH, scratch_types=scratch,
                   compiler_params=pltpu.CompilerParams(
                       use_tc_tiling_on_sc=False))
    return fn(q16, coef, u2d, v2d, zeros16)[0]


def kernel(atom_ftr, bond_ftr, massive, edge_index, mol_ids,
           W_init_v, b_init_v, W_init_e, b_init_e, W_p, W_q, W_msg,
           W_upd_v, W_upd_e, W_ham, W_att, W_fp, W_c1, b_c1, W_c2, b_c2):
    def pad_cols(w, cols):
        return jnp.pad(w, ((0, 0), (0, cols - w.shape[1])))

    def pad_rows(w, rows):
        return jnp.pad(w, ((0, rows - w.shape[0]), (0, 0)))

    def pad_nodes(x):
        return jnp.pad(x, ((0, NP - N), (0, 0)))

    def pad_edges(x):
        return jnp.pad(x, ((0, E2 - E), (0, 0)))

    u2d = jnp.concatenate(
        [edge_index[0].astype(jnp.int32),
         jnp.full((E2 - E,), N, jnp.int32)]).reshape(E2 // CHUNK, CHUNK)
    v2d = jnp.concatenate(
        [edge_index[1].astype(jnp.int32),
         jnp.full((E2 - E,), N, jnp.int32)]).reshape(E2 // CHUNK, CHUNK)
    zeros64 = jnp.zeros((NP, ME), jnp.float32)
    zeros16 = jnp.zeros((NP, 16), jnp.float32)
    mass_pad = jnp.pad(massive, ((0, NP - N), (0, 0)), constant_values=1.0)

    wp16 = pad_cols(W_p, 16)
    wq16 = pad_cols(W_q, 16)
    wd16 = [pad_rows(W_msg[i][2 * HV + HE:], 16) for i in range(2)]
    wmu = [W_msg[i][:HV] for i in range(2)]
    wmv = [W_msg[i][HV:2 * HV] for i in range(2)]
    whu = [W_ham[i][:HV] for i in range(2)]
    whv = [W_ham[i][HV:2 * HV] for i in range(2)]

    hv, p16, q16, tu, tv = _init_nodes(
        atom_ftr, W_init_v, b_init_v[None], wp16, wq16,
        wmu[0], wmv[0], wd16[0], whu[0], whv[0])
    p16 = pad_nodes(p16)
    q16 = pad_nodes(q16)
    he, mhe, ce = _init_edges(
        bond_ftr, W_init_e, b_init_e[None],
        W_msg[0][2 * HV:2 * HV + HE], W_ham[0][2 * HV:])

    for i in range(2):
        mhe_p = pad_edges(mhe)
        ce_p = pad_edges(ce)
        m, coef, agg2 = _sc_edge_msg(pad_nodes(tu), pad_nodes(tv), mhe_p,
                                     ce_p, u2d, v2d, zeros64,
                                     write_m=(i == 0))
        t_hv = _upd_v(hv, agg2[:, :N], W_upd_v[i][:HV], W_upd_v[i][HV:])
        if i == 0:
            mhe, ce = _upd_e_proj(he, m[:E], W_upd_e[0][:HE],
                                  W_upd_e[0][HE:],
                                  W_msg[1][2 * HV:2 * HV + HE],
                                  W_ham[1][2 * HV:])
        for _ in range(4):
            f2 = _sc_ham_f(q16, coef, u2d, v2d, zeros16)
            q16, p16 = _ham_update(q16, p16, f2, mass_pad)
        hv = t_hv
        if i == 0:
            tu, tv = _node_tables(hv, q16[:N], wmu[1], wmv[1], wd16[1],
                                  whu[1], whv[1])

    fp, conf8 = _readout(
        hv, p16[:N], q16[:N], mol_ids[:, None].astype(jnp.int32),
        W_fp, W_att[:HV], pad_rows(W_att[HV:HV + PQ], 16),
        pad_rows(W_att[HV + PQ:], 16), pad_rows(W_c1, 16), b_c1[None],
        pad_cols(W_c2, 8), pad_cols(b_c2[None], 8))
    return (fp, conf8[:, :PQ])
